# Initial kernel scaffold; baseline (speedup 1.0000x reference)
import functools

import jax
import jax.numpy as jnp
from jax import lax
from jax.experimental import pallas as pl
from jax.experimental.pallas import tpu as pltpu
from jax.experimental.pallas import tpu_sc as plsc

MASKID = 0
LANES = 128
NBUF = 2


@functools.lru_cache(maxsize=None)
def _build(n_rows, vocab, dim):
    info = plsc.get_sparse_core_info()
    nw = info.num_cores * info.num_subcores
    rows_w = n_rows // nw
    assert rows_w * nw == n_rows

    mesh = plsc.VectorSubcoreMesh(core_axis_name="c", subcore_axis_name="s")

    @functools.partial(
        pl.kernel,
        mesh=mesh,
        out_type=(
            jax.ShapeDtypeStruct((n_rows * LANES, dim), jnp.float32),
            jax.ShapeDtypeStruct((n_rows, LANES), jnp.int32),
        ),
        scratch_types=[
            pltpu.VMEM((rows_w, LANES), jnp.int32),
            pltpu.VMEM((rows_w, LANES), jnp.int32),
            [pltpu.VMEM((LANES, dim), jnp.float32) for _ in range(NBUF)],
            [pltpu.SemaphoreType.DMA for _ in range(NBUF)],
        ],
    )
    def emb(x_hbm, table_hbm, out_hbm, mask_hbm, idx_v, mask_v, bufs, sems):
        wid = lax.axis_index("s") * info.num_cores + lax.axis_index("c")
        row0 = wid * rows_w
        out0 = row0 * LANES

        pltpu.sync_copy(x_hbm.at[pl.ds(row0, rows_w)], idx_v)

        for b in range(NBUF):
            pltpu.async_copy(table_hbm.at[idx_v.at[b]], bufs[b], sems[b])

        def mask_body(i, _):
            for j in range(LANES // 16):
                v = idx_v[i, pl.ds(j * 16, 16)]
                mask_v[i, pl.ds(j * 16, 16)] = jnp.where(
                    v != MASKID, 1, 0
                ).astype(jnp.int32)
            return 0

        lax.fori_loop(0, rows_w, mask_body, 0)
        pltpu.sync_copy(mask_v, mask_hbm.at[pl.ds(row0, rows_w)])

        def body(g, _):
            for b in range(NBUF):
                c = g * NBUF + b
                pltpu.make_async_copy(
                    table_hbm.at[idx_v.at[c]], bufs[b], sems[b]
                ).wait()
                pltpu.sync_copy(
                    bufs[b], out_hbm.at[pl.ds(out0 + c * LANES, LANES)]
                )

                @pl.when(c + NBUF < rows_w)
                def _():
                    pltpu.async_copy(
                        table_hbm.at[idx_v.at[c + NBUF]], bufs[b], sems[b]
                    )

            return 0

        lax.fori_loop(0, rows_w // NBUF, body, 0)

    return emb


def kernel(x, table):
    bsz, seq = x.shape
    vocab, dim = table.shape
    total = bsz * seq
    n_rows = total // LANES
    xf = x.reshape(n_rows, LANES).astype(jnp.int32)
    emb = _build(n_rows, vocab, dim)
    out, mask = emb(xf, table)
    return out.reshape(bsz, seq, dim), mask.reshape(bsz, seq)


# speedup vs baseline: 3.3185x; 3.3185x over previous
import functools

import jax
import jax.numpy as jnp
from jax import lax
from jax.experimental import pallas as pl
from jax.experimental.pallas import tpu as pltpu
from jax.experimental.pallas import tpu_sc as plsc

MASKID = 0
CHUNK = 128
NBUF = 2


@functools.lru_cache(maxsize=None)
def _build(total, vocab, dim):
    info = plsc.get_sparse_core_info()
    nw = info.num_cores * info.num_subcores
    per_w = total // nw
    n_chunks = per_w // CHUNK
    assert per_w * nw == total and n_chunks * CHUNK == per_w

    mesh = plsc.VectorSubcoreMesh(core_axis_name="c", subcore_axis_name="s")

    @functools.partial(
        pl.kernel,
        mesh=mesh,
        out_type=(
            jax.ShapeDtypeStruct((total, dim), jnp.float32),
            jax.ShapeDtypeStruct((total,), jnp.int32),
        ),
        scratch_types=[
            pltpu.VMEM((per_w,), jnp.int32),
            pltpu.VMEM((per_w,), jnp.int32),
            [pltpu.VMEM((CHUNK, dim), jnp.float32) for _ in range(NBUF)],
            [pltpu.SemaphoreType.DMA for _ in range(NBUF)],
        ],
    )
    def emb(x_hbm, table_hbm, out_hbm, mask_hbm, idx_v, mask_v, bufs, sems):
        wid = lax.axis_index("s") * info.num_cores + lax.axis_index("c")
        base = wid * per_w

        pltpu.sync_copy(x_hbm.at[pl.ds(base, per_w)], idx_v)

        for b in range(NBUF):
            pltpu.async_copy(
                table_hbm.at[idx_v.at[pl.ds(b * CHUNK, CHUNK)]],
                bufs[b], sems[b],
            )

        def mask_body(i, _):
            for j in range(CHUNK // 16):
                o = i * CHUNK + j * 16
                v = idx_v[pl.ds(o, 16)]
                mask_v[pl.ds(o, 16)] = jnp.where(v != MASKID, 1, 0).astype(
                    jnp.int32
                )
            return 0

        lax.fori_loop(0, n_chunks, mask_body, 0)
        pltpu.sync_copy(mask_v, mask_hbm.at[pl.ds(base, per_w)])

        def body(g, _):
            for b in range(NBUF):
                c = g * NBUF + b
                pltpu.make_async_copy(
                    table_hbm.at[idx_v.at[pl.ds(c * CHUNK, CHUNK)]],
                    bufs[b], sems[b],
                ).wait()
                pltpu.sync_copy(
                    bufs[b], out_hbm.at[pl.ds(base + c * CHUNK, CHUNK)]
                )

                @pl.when(c + NBUF < n_chunks)
                def _():
                    pltpu.async_copy(
                        table_hbm.at[idx_v.at[pl.ds((c + NBUF) * CHUNK, CHUNK)]],
                        bufs[b], sems[b],
                    )

            return 0

        lax.fori_loop(0, n_chunks // NBUF, body, 0)

    return emb


def kernel(x, table):
    bsz, seq = x.shape
    vocab, dim = table.shape
    total = bsz * seq
    xf = x.reshape(total).astype(jnp.int32)
    emb = _build(total, vocab, dim)
    out, mask = emb(xf, table)
    return out.reshape(bsz, seq, dim), mask.reshape(bsz, seq)


# speedup vs baseline: 3.3391x; 1.0062x over previous
import functools

import jax
import jax.numpy as jnp
from jax import lax
from jax.experimental import pallas as pl
from jax.experimental.pallas import tpu as pltpu
from jax.experimental.pallas import tpu_sc as plsc

MASKID = 0
CHUNK = 128
NBUF = 5
LOOKAHEAD = 3


@functools.lru_cache(maxsize=None)
def _build(total, vocab, dim):
    info = plsc.get_sparse_core_info()
    nw = info.num_cores * info.num_subcores
    per_w = total // nw
    n_chunks = per_w // CHUNK
    assert per_w * nw == total and n_chunks * CHUNK == per_w
    assert n_chunks % NBUF == 0 and LOOKAHEAD < NBUF

    mesh = plsc.VectorSubcoreMesh(core_axis_name="c", subcore_axis_name="s")

    @functools.partial(
        pl.kernel,
        mesh=mesh,
        out_type=(
            jax.ShapeDtypeStruct((total, dim), jnp.float32),
            jax.ShapeDtypeStruct((total,), jnp.int32),
        ),
        scratch_types=[
            pltpu.VMEM((per_w,), jnp.int32),
            pltpu.VMEM((per_w,), jnp.int32),
            [pltpu.VMEM((CHUNK, dim), jnp.float32) for _ in range(NBUF)],
            [pltpu.SemaphoreType.DMA for _ in range(NBUF)],
            [pltpu.SemaphoreType.DMA for _ in range(NBUF)],
        ],
    )
    def emb(x_hbm, table_hbm, out_hbm, mask_hbm,
            idx_v, mask_v, bufs, isems, osems):
        wid = lax.axis_index("s") * info.num_cores + lax.axis_index("c")
        base = wid * per_w

        pltpu.sync_copy(x_hbm.at[pl.ds(base, per_w)], idx_v)

        def gather(c, b):
            pltpu.async_copy(
                table_hbm.at[idx_v.at[pl.ds(c * CHUNK, CHUNK)]],
                bufs[b], isems[b],
            )

        def drain_in(c, b):
            pltpu.make_async_copy(
                table_hbm.at[idx_v.at[pl.ds(c * CHUNK, CHUNK)]],
                bufs[b], isems[b],
            ).wait()

        def copyout(c, b):
            pltpu.async_copy(
                bufs[b], out_hbm.at[pl.ds(base + c * CHUNK, CHUNK)], osems[b]
            )

        def drain_out(c, b):
            pltpu.make_async_copy(
                bufs[b], out_hbm.at[pl.ds(base + c * CHUNK, CHUNK)], osems[b]
            ).wait()

        for c in range(LOOKAHEAD):
            gather(c, c % NBUF)

        def mask_body(i, _):
            for j in range(CHUNK // 16):
                o = i * CHUNK + j * 16
                v = idx_v[pl.ds(o, 16)]
                mask_v[pl.ds(o, 16)] = jnp.where(v != MASKID, 1, 0).astype(
                    jnp.int32
                )
            return 0

        lax.fori_loop(0, n_chunks, mask_body, 0)
        pltpu.sync_copy(mask_v, mask_hbm.at[pl.ds(base, per_w)])

        for b in range(NBUF):
            c = b
            drain_in(c, b)
            copyout(c, b)
            cg = c + LOOKAHEAD
            bg = cg % NBUF
            if cg >= NBUF:
                drain_out(cg - NBUF, bg)
            gather(cg, bg)

        def body(g, _):
            for b in range(NBUF):
                c = g * NBUF + b
                drain_in(c, b)
                copyout(c, b)
                cg = c + LOOKAHEAD
                bg = (b + LOOKAHEAD) % NBUF

                @pl.when(cg < n_chunks)
                def _():
                    drain_out(cg - NBUF, bg)
                    gather(cg, bg)

            return 0

        lax.fori_loop(1, n_chunks // NBUF, body, 0)

        for b in range(NBUF):
            drain_out(n_chunks - NBUF + b, b)

    return emb


def kernel(x, table):
    bsz, seq = x.shape
    vocab, dim = table.shape
    total = bsz * seq
    xf = x.reshape(total).astype(jnp.int32)
    emb = _build(total, vocab, dim)
    out, mask = emb(xf, table)
    return out.reshape(bsz, seq, dim), mask.reshape(bsz, seq)
